# Initial kernel scaffold; baseline (speedup 1.0000x reference)
import jax
import jax.numpy as jnp
from jax.experimental import pallas as pl


def kernel(x, edge_index, W_l, W_r, att, bias):
    raise NotImplementedError("write your pallas kernel here")


# speedup vs baseline: 15.8034x; 15.8034x over previous
import functools

import jax
import jax.numpy as jnp
from jax import lax
from jax.experimental import pallas as pl
from jax.experimental.pallas import tpu as pltpu
from jax.experimental.pallas import tpu_sc as plsc

NEG_SLOPE = 0.2
CB = 64
NCORES = 2
NSUB = 16
NTILES = NCORES * NSUB


def _proj_body(x_ref, wl_ref, wr_ref, xl_ref, xr_ref):
    xv = x_ref[...]
    xl_ref[...] = jnp.dot(xv, wl_ref[...], preferred_element_type=jnp.float32)
    xr_ref[...] = jnp.dot(xv, wr_ref[...], preferred_element_type=jnp.float32)


def _fin_body(n0_ref, n1_ref, d0_ref, d1_ref, b_ref, s_ref, o_ref):
    num = n0_ref[...] + n1_ref[...]
    den = d0_ref[...] + d1_ref[...]
    recip = 1.0 / (den + 1e-16)
    denx = jnp.dot(recip, s_ref[...], preferred_element_type=jnp.float32)
    o_ref[...] = jnp.maximum(num * denx + b_ref[...][0:1, :], 0.0)


def _make_sc_kernel(npad, epad, H, C):
    HC = H * C
    ept = epad // NTILES
    nch = ept // CB
    rps = npad // NSUB

    mesh = plsc.VectorSubcoreMesh(
        core_axis_name="c", subcore_axis_name="s",
        num_cores=NCORES, num_subcores=NSUB)

    @functools.partial(
        pl.kernel,
        out_type=[
            jax.ShapeDtypeStruct((npad, HC), jnp.float32),
            jax.ShapeDtypeStruct((npad, HC), jnp.float32),
            jax.ShapeDtypeStruct((npad // 8, 128), jnp.float32),
            jax.ShapeDtypeStruct((npad // 8, 128), jnp.float32),
        ],
        mesh=mesh,
        compiler_params=pltpu.CompilerParams(needs_layout_passes=False),
        scratch_types=[
            pltpu.VMEM((HC,), jnp.float32),
            pltpu.VMEM((CB,), jnp.int32),
            pltpu.VMEM((CB,), jnp.int32),
            pltpu.VMEM((CB,), jnp.int32),
            pltpu.VMEM((CB, HC), jnp.float32),
            pltpu.VMEM((CB, HC), jnp.float32),
            pltpu.VMEM((16, CB), jnp.float32),
            pltpu.VMEM_SHARED((npad, HC), jnp.float32),
            pltpu.VMEM_SHARED((npad // 8, 128), jnp.float32),
            pltpu.SemaphoreType.DMA,
        ],
    )
    def sc_kernel(xl_h, xr_h, src_h, dst_h, att_h,
                  num0_o, num1_o, den0_o, den1_o,
                  att_v, src_v, dst_v, dstpk_v, xlr, xrr, wtmp, num_sh, den_sh,
                  sem):
        cid = lax.axis_index("c")
        sid = lax.axis_index("s")
        r0 = sid * rps
        pr0 = sid * (rps // 8)
        iota = lax.iota(jnp.int32, 16)
        zero16 = jnp.zeros((16,), jnp.float32)

        pltpu.sync_copy(att_h, att_v)
        for r in range(8):
            for c8 in range(HC // 16):
                xlr[r, pl.ds(c8 * 16, 16)] = zero16

        def zinit(t, carry):
            pltpu.sync_copy(xlr.at[pl.ds(0, 8)], num_sh.at[pl.ds(r0 + t * 8, 8)])
            return carry
        lax.fori_loop(0, rps // 8, zinit, 0)

        def zinit2(t, carry):
            pltpu.sync_copy(xlr.at[pl.ds(0, 8)], den_sh.at[pl.ds(pr0 + t * 8, 8)])
            return carry
        lax.fori_loop(0, rps // 64, zinit2, 0)

        plsc.subcore_barrier()
        tid = cid * NSUB + sid
        ebase0 = tid * ept

        def chunk(gc, carry):
            eb = ebase0 + gc * CB
            pltpu.sync_copy(src_h.at[pl.ds(eb, CB)], src_v)
            pltpu.sync_copy(dst_h.at[pl.ds(eb, CB)], dst_v)
            pltpu.async_copy(xl_h.at[src_v], xlr, sem).wait()
            pltpu.async_copy(xr_h.at[dst_v], xrr, sem).wait()
            for g16 in range(CB // 16):
                dv = dst_v[pl.ds(g16 * 16, 16)]
                dstpk_v[pl.ds(g16 * 16, 16)] = lax.shift_right_logical(dv, 3)
            for g16 in range(CB // 16):
                ridx = iota + (g16 * 16)
                for h in range(H):
                    def ch(c4, acc, _h=h):
                        for dc in range(4):
                            c = _h * C + c4 * 4 + dc
                            colv = jnp.full((16,), c, jnp.int32)
                            xlv = plsc.load_gather(xlr, [ridx, colv])
                            xrv = plsc.load_gather(xrr, [ridx, colv])
                            av = plsc.load_gather(att_v, [colv])
                            v = xlv + xrv
                            v = jnp.maximum(v, NEG_SLOPE * v)
                            acc = acc + av * v
                        return acc
                    acc = lax.fori_loop(0, C // 4, ch, zero16)
                    wtmp[h, pl.ds(g16 * 16, 16)] = jnp.exp(acc)

            def rowp(r, carry):
                rv = jnp.full((16,), r, jnp.int32)
                roww = zero16
                for h in range(H):
                    ws = plsc.load_gather(
                        wtmp, [jnp.full((16,), h, jnp.int32), rv])
                    for cc in range(C // 16):
                        sl = pl.ds(h * C + cc * 16, 16)
                        xlr[r, sl] = xlr[r, sl] * ws
                    roww = jnp.where(iota == h, ws, roww)
                dsplat = plsc.load_gather(dst_v, [rv])
                off = jnp.max((dsplat & 7) * 16)
                for s8 in range(8):
                    xrr[r, pl.ds(s8 * 16, 16)] = zero16
                xrr[r, pl.ds(off, 16)] = roww
                return carry
            lax.fori_loop(0, CB, rowp, 0)

            pltpu.sync_copy(xlr, num_sh.at[dst_v], add=True)
            pltpu.sync_copy(xrr, den_sh.at[dstpk_v], add=True)
            return carry

        lax.fori_loop(0, nch, chunk, 0)
        plsc.subcore_barrier()

        def copy_out(num_o, den_o):
            def cp(t, carry):
                rr = r0 + t * CB
                prr = pr0 + t * 8
                pltpu.sync_copy(num_sh.at[pl.ds(rr, CB)], xlr)
                pltpu.sync_copy(xlr, num_o.at[pl.ds(rr, CB)])
                pltpu.sync_copy(den_sh.at[pl.ds(prr, 8)], xrr.at[pl.ds(0, 8)])
                pltpu.sync_copy(xrr.at[pl.ds(0, 8)], den_o.at[pl.ds(prr, 8)])
                return carry
            lax.fori_loop(0, rps // CB, cp, 0)

        @pl.when(cid == 0)
        def _():
            copy_out(num0_o, den0_o)

        @pl.when(cid == 1)
        def _():
            copy_out(num1_o, den1_o)

    return sc_kernel


def kernel(x, edge_index, W_l, W_r, att, bias):
    n, in_dim = x.shape
    H, C = att.shape
    HC = H * C
    e = edge_index.shape[1]
    etot = e + n
    npad = ((n + 1 + 1023) // 1024) * 1024
    epad = ((etot + CB * NTILES - 1) // (CB * NTILES)) * (CB * NTILES)

    x_pad = jnp.zeros((npad, in_dim), jnp.float32).at[:n].set(x)
    loop = jnp.arange(n, dtype=jnp.int32)
    padi = jnp.full((epad - etot,), n, dtype=jnp.int32)
    src_all = jnp.concatenate([edge_index[0], loop, padi])
    dst_all = jnp.concatenate([edge_index[1], loop, padi])
    att_flat = att.reshape(HC)

    nblk = 8
    brows = npad // nblk
    xl, xr = pl.pallas_call(
        _proj_body,
        grid=(nblk,),
        in_specs=[
            pl.BlockSpec((brows, in_dim), lambda i: (i, 0)),
            pl.BlockSpec((in_dim, HC), lambda i: (0, 0)),
            pl.BlockSpec((in_dim, HC), lambda i: (0, 0)),
        ],
        out_specs=[
            pl.BlockSpec((brows, HC), lambda i: (i, 0)),
            pl.BlockSpec((brows, HC), lambda i: (i, 0)),
        ],
        out_shape=[
            jax.ShapeDtypeStruct((npad, HC), jnp.float32),
            jax.ShapeDtypeStruct((npad, HC), jnp.float32),
        ],
    )(x_pad, W_l, W_r)

    sc_kernel = _make_sc_kernel(npad, epad, H, C)
    num0, num1, den0, den1 = sc_kernel(
        xl, xr, src_all, dst_all, att_flat)
    den0 = den0.reshape(npad, 16)
    den1 = den1.reshape(npad, 16)

    sel = jnp.zeros((16, HC), jnp.float32).at[:H].set(
        jnp.repeat(jnp.eye(H, dtype=jnp.float32), C, axis=1))
    bias_b = jnp.broadcast_to(bias.reshape(1, HC), (8, HC))
    out = pl.pallas_call(
        _fin_body,
        grid=(nblk,),
        in_specs=[
            pl.BlockSpec((brows, HC), lambda i: (i, 0)),
            pl.BlockSpec((brows, HC), lambda i: (i, 0)),
            pl.BlockSpec((brows, 16), lambda i: (i, 0)),
            pl.BlockSpec((brows, 16), lambda i: (i, 0)),
            pl.BlockSpec((8, HC), lambda i: (0, 0)),
            pl.BlockSpec((16, HC), lambda i: (0, 0)),
        ],
        out_specs=pl.BlockSpec((brows, HC), lambda i: (i, 0)),
        out_shape=jax.ShapeDtypeStruct((npad, HC), jnp.float32),
    )(num0, num1, den0, den1, bias_b, sel)
    return out[:n]


# speedup vs baseline: 21.6038x; 1.3670x over previous
import functools

import jax
import jax.numpy as jnp
from jax import lax
from jax.experimental import pallas as pl
from jax.experimental.pallas import tpu as pltpu
from jax.experimental.pallas import tpu_sc as plsc

NEG_SLOPE = 0.2
CB = 64
NCORES = 2
NSUB = 16
NTILES = NCORES * NSUB


def _proj_body(x_ref, wl_ref, wr_ref, xl_ref, xr_ref):
    xv = x_ref[...]
    xl_ref[...] = jnp.dot(xv, wl_ref[...], preferred_element_type=jnp.float32)
    xr_ref[...] = jnp.dot(xv, wr_ref[...], preferred_element_type=jnp.float32)


def _fin_body(n0_ref, n1_ref, d0_ref, d1_ref, b_ref, s_ref, o_ref):
    num = n0_ref[...] + n1_ref[...]
    den = d0_ref[...] + d1_ref[...]
    recip = 1.0 / (den + 1e-16)
    denx = jnp.dot(recip, s_ref[...], preferred_element_type=jnp.float32)
    o_ref[...] = jnp.maximum(num * denx + b_ref[...][0:1, :], 0.0)


def _make_sc_kernel(npad, epad, H, C):
    HC = H * C
    ept = epad // NTILES
    nch = ept // CB
    rps = npad // NSUB

    mesh = plsc.VectorSubcoreMesh(
        core_axis_name="c", subcore_axis_name="s",
        num_cores=NCORES, num_subcores=NSUB)

    @functools.partial(
        pl.kernel,
        out_type=[
            jax.ShapeDtypeStruct((npad, HC), jnp.float32),
            jax.ShapeDtypeStruct((npad, HC), jnp.float32),
            jax.ShapeDtypeStruct((npad // 8, 128), jnp.float32),
            jax.ShapeDtypeStruct((npad // 8, 128), jnp.float32),
        ],
        mesh=mesh,
        compiler_params=pltpu.CompilerParams(needs_layout_passes=False),
        scratch_types=[
            pltpu.VMEM((HC,), jnp.float32),
            pltpu.VMEM((CB,), jnp.int32),
            pltpu.VMEM((CB,), jnp.int32),
            pltpu.VMEM((CB,), jnp.int32),
            pltpu.VMEM((CB, HC), jnp.float32),
            pltpu.VMEM((CB, HC), jnp.float32),
            pltpu.VMEM_SHARED((npad, HC), jnp.float32),
            pltpu.VMEM_SHARED((npad // 8, 128), jnp.float32),
            pltpu.SemaphoreType.DMA,
        ],
    )
    def sc_kernel(xl_h, xr_h, src_h, dst_h, att_h,
                  num0_o, num1_o, den0_o, den1_o,
                  att_v, src_v, dst_v, dstpk_v, xlr, xrr, num_sh, den_sh,
                  sem):
        cid = lax.axis_index("c")
        sid = lax.axis_index("s")
        r0 = sid * rps
        pr0 = sid * (rps // 8)
        iota = lax.iota(jnp.int32, 16)
        zero16 = jnp.zeros((16,), jnp.float32)

        pltpu.sync_copy(att_h, att_v)
        for r in range(8):
            for c8 in range(HC // 16):
                xlr[r, pl.ds(c8 * 16, 16)] = zero16

        def zinit(t, carry):
            pltpu.sync_copy(xlr.at[pl.ds(0, 8)], num_sh.at[pl.ds(r0 + t * 8, 8)])
            return carry
        lax.fori_loop(0, rps // 8, zinit, 0)

        def zinit2(t, carry):
            pltpu.sync_copy(xlr.at[pl.ds(0, 8)], den_sh.at[pl.ds(pr0 + t * 8, 8)])
            return carry
        lax.fori_loop(0, rps // 64, zinit2, 0)

        plsc.subcore_barrier()
        tid = cid * NSUB + sid
        ebase0 = tid * ept
        nv = HC // 16
        attv = [att_v[pl.ds(k * 16, 16)] for k in range(nv)]

        def chunk(gc, attv):
            eb = ebase0 + gc * CB
            pltpu.sync_copy(src_h.at[pl.ds(eb, CB)], src_v)
            pltpu.sync_copy(dst_h.at[pl.ds(eb, CB)], dst_v)
            pltpu.async_copy(xl_h.at[src_v], xlr, sem).wait()
            pltpu.async_copy(xr_h.at[dst_v], xrr, sem).wait()
            for g16 in range(CB // 16):
                dv = dst_v[pl.ds(g16 * 16, 16)]
                dstpk_v[pl.ds(g16 * 16, 16)] = lax.shift_right_logical(dv, 3)

            kph = C // 16
            def edge(r, carry):
                rv = jnp.full((16,), r, jnp.int32)
                roww = zero16
                for h in range(H):
                    acc = zero16
                    xsl = []
                    for k in range(kph):
                        sl = pl.ds((h * kph + k) * 16, 16)
                        xv = xlr[r, sl]
                        xsl.append(xv)
                        v = xv + xrr[r, sl]
                        v = jnp.maximum(v, NEG_SLOPE * v)
                        acc = acc + attv[h * kph + k] * v
                    w = jnp.exp(jnp.full((16,), jnp.sum(acc)))
                    for k in range(kph):
                        sl = pl.ds((h * kph + k) * 16, 16)
                        xlr[r, sl] = xsl[k] * w
                    roww = jnp.where(iota == h, w, roww)
                dsplat = plsc.load_gather(dst_v, [rv])
                for s8 in range(nv):
                    xrr[r, pl.ds(s8 * 16, 16)] = zero16
                colv = (dsplat & 7) * 16 + iota
                plsc.store_scatter(xrr, [rv, colv], roww)
                return carry
            lax.fori_loop(0, CB, edge, 0)

            pltpu.sync_copy(xlr, num_sh.at[dst_v], add=True)
            pltpu.sync_copy(xrr, den_sh.at[dstpk_v], add=True)
            return attv

        lax.fori_loop(0, nch, chunk, attv)
        plsc.subcore_barrier()

        def copy_out(num_o, den_o):
            def cp(t, carry):
                rr = r0 + t * CB
                prr = pr0 + t * 8
                pltpu.sync_copy(num_sh.at[pl.ds(rr, CB)], xlr)
                pltpu.sync_copy(xlr, num_o.at[pl.ds(rr, CB)])
                pltpu.sync_copy(den_sh.at[pl.ds(prr, 8)], xrr.at[pl.ds(0, 8)])
                pltpu.sync_copy(xrr.at[pl.ds(0, 8)], den_o.at[pl.ds(prr, 8)])
                return carry
            lax.fori_loop(0, rps // CB, cp, 0)

        @pl.when(cid == 0)
        def _():
            copy_out(num0_o, den0_o)

        @pl.when(cid == 1)
        def _():
            copy_out(num1_o, den1_o)

    return sc_kernel


def kernel(x, edge_index, W_l, W_r, att, bias):
    n, in_dim = x.shape
    H, C = att.shape
    HC = H * C
    e = edge_index.shape[1]
    etot = e + n
    npad = ((n + 1 + 1023) // 1024) * 1024
    epad = ((etot + CB * NTILES - 1) // (CB * NTILES)) * (CB * NTILES)

    x_pad = jnp.zeros((npad, in_dim), jnp.float32).at[:n].set(x)
    loop = jnp.arange(n, dtype=jnp.int32)
    padi = jnp.full((epad - etot,), n, dtype=jnp.int32)
    src_all = jnp.concatenate([edge_index[0], loop, padi])
    dst_all = jnp.concatenate([edge_index[1], loop, padi])
    att_flat = att.reshape(HC)

    nblk = 8
    brows = npad // nblk
    xl, xr = pl.pallas_call(
        _proj_body,
        grid=(nblk,),
        in_specs=[
            pl.BlockSpec((brows, in_dim), lambda i: (i, 0)),
            pl.BlockSpec((in_dim, HC), lambda i: (0, 0)),
            pl.BlockSpec((in_dim, HC), lambda i: (0, 0)),
        ],
        out_specs=[
            pl.BlockSpec((brows, HC), lambda i: (i, 0)),
            pl.BlockSpec((brows, HC), lambda i: (i, 0)),
        ],
        out_shape=[
            jax.ShapeDtypeStruct((npad, HC), jnp.float32),
            jax.ShapeDtypeStruct((npad, HC), jnp.float32),
        ],
    )(x_pad, W_l, W_r)

    sc_kernel = _make_sc_kernel(npad, epad, H, C)
    num0, num1, den0, den1 = sc_kernel(
        xl, xr, src_all, dst_all, att_flat)
    den0 = den0.reshape(npad, 16)
    den1 = den1.reshape(npad, 16)

    sel = jnp.zeros((16, HC), jnp.float32).at[:H].set(
        jnp.repeat(jnp.eye(H, dtype=jnp.float32), C, axis=1))
    bias_b = jnp.broadcast_to(bias.reshape(1, HC), (8, HC))
    out = pl.pallas_call(
        _fin_body,
        grid=(nblk,),
        in_specs=[
            pl.BlockSpec((brows, HC), lambda i: (i, 0)),
            pl.BlockSpec((brows, HC), lambda i: (i, 0)),
            pl.BlockSpec((brows, 16), lambda i: (i, 0)),
            pl.BlockSpec((brows, 16), lambda i: (i, 0)),
            pl.BlockSpec((8, HC), lambda i: (0, 0)),
            pl.BlockSpec((16, HC), lambda i: (0, 0)),
        ],
        out_specs=pl.BlockSpec((brows, HC), lambda i: (i, 0)),
        out_shape=jax.ShapeDtypeStruct((npad, HC), jnp.float32),
    )(num0, num1, den0, den1, bias_b, sel)
    return out[:n]


# speedup vs baseline: 41.2239x; 1.9082x over previous
import functools

import jax
import jax.numpy as jnp
from jax import lax
from jax.experimental import pallas as pl
from jax.experimental.pallas import tpu as pltpu
from jax.experimental.pallas import tpu_sc as plsc

NEG_SLOPE = 0.2
CB = 64
NCORES = 2
NSUB = 16
NTILES = NCORES * NSUB


def _proj_body(x_ref, wl_ref, wr_ref, xl_ref, xr_ref):
    xv = x_ref[...]
    xl_ref[...] = jnp.dot(xv, wl_ref[...], preferred_element_type=jnp.float32)
    xr_ref[...] = jnp.dot(xv, wr_ref[...], preferred_element_type=jnp.float32)


def _fin_body(n0_ref, n1_ref, d0_ref, d1_ref, b_ref, s_ref, o_ref):
    num = n0_ref[...] + n1_ref[...]
    den = d0_ref[...] + d1_ref[...]
    recip = 1.0 / (den + 1e-16)
    denx = jnp.dot(recip, s_ref[...], preferred_element_type=jnp.float32)
    o_ref[...] = jnp.maximum(num * denx + b_ref[...][0:1, :], 0.0)


def _make_sc_kernel(npad, epad, H, C):
    HC = H * C
    ept = epad // NTILES
    nch = ept // CB
    rps = npad // NSUB

    mesh = plsc.VectorSubcoreMesh(
        core_axis_name="c", subcore_axis_name="s",
        num_cores=NCORES, num_subcores=NSUB)

    @functools.partial(
        pl.kernel,
        out_type=[
            jax.ShapeDtypeStruct((npad, HC), jnp.float32),
            jax.ShapeDtypeStruct((npad, HC), jnp.float32),
            jax.ShapeDtypeStruct((npad // 8, 128), jnp.float32),
            jax.ShapeDtypeStruct((npad // 8, 128), jnp.float32),
        ],
        mesh=mesh,
        compiler_params=pltpu.CompilerParams(needs_layout_passes=False),
        scratch_types=[
            pltpu.VMEM((HC,), jnp.float32),
            pltpu.VMEM((CB,), jnp.int32),
            pltpu.VMEM((CB,), jnp.int32),
            pltpu.VMEM((CB,), jnp.int32),
            pltpu.VMEM((CB, HC), jnp.float32),
            pltpu.VMEM((CB, HC), jnp.float32),
            pltpu.VMEM_SHARED((npad, HC), jnp.float32),
            pltpu.VMEM_SHARED((npad // 8, 128), jnp.float32),
            pltpu.SemaphoreType.DMA,
        ],
    )
    def sc_kernel(xl_h, xr_h, src_h, dst_h, att_h,
                  num0_o, num1_o, den0_o, den1_o,
                  att_v, src_v, dst_v, dstpk_v, xlr, xrr, num_sh, den_sh,
                  sem):
        cid = lax.axis_index("c")
        sid = lax.axis_index("s")
        r0 = sid * rps
        pr0 = sid * (rps // 8)
        iota = lax.iota(jnp.int32, 16)
        zero16 = jnp.zeros((16,), jnp.float32)

        pltpu.sync_copy(att_h, att_v)
        for r in range(8):
            for c8 in range(HC // 16):
                xlr[r, pl.ds(c8 * 16, 16)] = zero16

        def zinit(t, carry):
            pltpu.sync_copy(xlr.at[pl.ds(0, 8)], num_sh.at[pl.ds(r0 + t * 8, 8)])
            return carry
        lax.fori_loop(0, rps // 8, zinit, 0)

        def zinit2(t, carry):
            pltpu.sync_copy(xlr.at[pl.ds(0, 8)], den_sh.at[pl.ds(pr0 + t * 8, 8)])
            return carry
        lax.fori_loop(0, rps // 64, zinit2, 0)

        plsc.subcore_barrier()
        tid = cid * NSUB + sid
        ebase0 = tid * ept
        nv = HC // 16
        attv = [att_v[pl.ds(k * 16, 16)] for k in range(nv)]

        def chunk(gc, attv):
            eb = ebase0 + gc * CB
            pltpu.sync_copy(src_h.at[pl.ds(eb, CB)], src_v)
            pltpu.sync_copy(dst_h.at[pl.ds(eb, CB)], dst_v)
            pltpu.async_copy(xl_h.at[src_v], xlr, sem).wait()
            pltpu.async_copy(xr_h.at[dst_v], xrr, sem).wait()
            for g16 in range(CB // 16):
                dv = dst_v[pl.ds(g16 * 16, 16)]
                dstpk_v[pl.ds(g16 * 16, 16)] = lax.shift_right_logical(dv, 3)

            kph = C // 16
            def edge(r, carry):
                rv = jnp.full((16,), r, jnp.int32)
                roww = zero16
                for h in range(H):
                    acc = zero16
                    xsl = []
                    for k in range(kph):
                        sl = pl.ds((h * kph + k) * 16, 16)
                        xv = xlr[r, sl]
                        xsl.append(xv)
                        v = xv + xrr[r, sl]
                        v = jnp.maximum(v, NEG_SLOPE * v)
                        acc = acc + attv[h * kph + k] * v
                    w = jnp.exp(jnp.full((16,), jnp.sum(acc)))
                    for k in range(kph):
                        sl = pl.ds((h * kph + k) * 16, 16)
                        xlr[r, sl] = xsl[k] * w
                    roww = jnp.where(iota == h, w, roww)
                dsplat = plsc.load_gather(dst_v, [rv])
                for s8 in range(nv):
                    xrr[r, pl.ds(s8 * 16, 16)] = zero16
                colv = (dsplat & 7) * 16 + iota
                plsc.store_scatter(xrr, [rv, colv], roww)
                return carry
            plsc.parallel_loop(0, CB, 1, unroll=4, carry=jnp.int32(0))(edge)

            pltpu.sync_copy(xlr, num_sh.at[dst_v], add=True)
            pltpu.sync_copy(xrr, den_sh.at[dstpk_v], add=True)
            return attv

        lax.fori_loop(0, nch, chunk, attv)
        plsc.subcore_barrier()

        def copy_out(num_o, den_o):
            def cp(t, carry):
                rr = r0 + t * CB
                prr = pr0 + t * 8
                pltpu.sync_copy(num_sh.at[pl.ds(rr, CB)], xlr)
                pltpu.sync_copy(xlr, num_o.at[pl.ds(rr, CB)])
                pltpu.sync_copy(den_sh.at[pl.ds(prr, 8)], xrr.at[pl.ds(0, 8)])
                pltpu.sync_copy(xrr.at[pl.ds(0, 8)], den_o.at[pl.ds(prr, 8)])
                return carry
            lax.fori_loop(0, rps // CB, cp, 0)

        @pl.when(cid == 0)
        def _():
            copy_out(num0_o, den0_o)

        @pl.when(cid == 1)
        def _():
            copy_out(num1_o, den1_o)

    return sc_kernel


def kernel(x, edge_index, W_l, W_r, att, bias):
    n, in_dim = x.shape
    H, C = att.shape
    HC = H * C
    e = edge_index.shape[1]
    etot = e + n
    npad = ((n + 1 + 1023) // 1024) * 1024
    epad = ((etot + CB * NTILES - 1) // (CB * NTILES)) * (CB * NTILES)

    x_pad = jnp.zeros((npad, in_dim), jnp.float32).at[:n].set(x)
    loop = jnp.arange(n, dtype=jnp.int32)
    padi = jnp.full((epad - etot,), n, dtype=jnp.int32)
    src_all = jnp.concatenate([edge_index[0], loop, padi])
    dst_all = jnp.concatenate([edge_index[1], loop, padi])
    att_flat = att.reshape(HC)

    nblk = 8
    brows = npad // nblk
    xl, xr = pl.pallas_call(
        _proj_body,
        grid=(nblk,),
        in_specs=[
            pl.BlockSpec((brows, in_dim), lambda i: (i, 0)),
            pl.BlockSpec((in_dim, HC), lambda i: (0, 0)),
            pl.BlockSpec((in_dim, HC), lambda i: (0, 0)),
        ],
        out_specs=[
            pl.BlockSpec((brows, HC), lambda i: (i, 0)),
            pl.BlockSpec((brows, HC), lambda i: (i, 0)),
        ],
        out_shape=[
            jax.ShapeDtypeStruct((npad, HC), jnp.float32),
            jax.ShapeDtypeStruct((npad, HC), jnp.float32),
        ],
    )(x_pad, W_l, W_r)

    sc_kernel = _make_sc_kernel(npad, epad, H, C)
    num0, num1, den0, den1 = sc_kernel(
        xl, xr, src_all, dst_all, att_flat)
    den0 = den0.reshape(npad, 16)
    den1 = den1.reshape(npad, 16)

    sel = jnp.zeros((16, HC), jnp.float32).at[:H].set(
        jnp.repeat(jnp.eye(H, dtype=jnp.float32), C, axis=1))
    bias_b = jnp.broadcast_to(bias.reshape(1, HC), (8, HC))
    out = pl.pallas_call(
        _fin_body,
        grid=(nblk,),
        in_specs=[
            pl.BlockSpec((brows, HC), lambda i: (i, 0)),
            pl.BlockSpec((brows, HC), lambda i: (i, 0)),
            pl.BlockSpec((brows, 16), lambda i: (i, 0)),
            pl.BlockSpec((brows, 16), lambda i: (i, 0)),
            pl.BlockSpec((8, HC), lambda i: (0, 0)),
            pl.BlockSpec((16, HC), lambda i: (0, 0)),
        ],
        out_specs=pl.BlockSpec((brows, HC), lambda i: (i, 0)),
        out_shape=jax.ShapeDtypeStruct((npad, HC), jnp.float32),
    )(num0, num1, den0, den1, bias_b, sel)
    return out[:n]


# speedup vs baseline: 42.0418x; 1.0198x over previous
import functools

import jax
import jax.numpy as jnp
from jax import lax
from jax.experimental import pallas as pl
from jax.experimental.pallas import tpu as pltpu
from jax.experimental.pallas import tpu_sc as plsc

NEG_SLOPE = 0.2
CB = 64
NCORES = 2
NSUB = 16
NTILES = NCORES * NSUB


def _proj_body(x_ref, wl_ref, wr_ref, xl_ref, xr_ref):
    xv = x_ref[...]
    xl_ref[...] = jnp.dot(xv, wl_ref[...], preferred_element_type=jnp.float32)
    xr_ref[...] = jnp.dot(xv, wr_ref[...], preferred_element_type=jnp.float32)


def _fin_body(n0_ref, n1_ref, d0_ref, d1_ref, b_ref, s_ref, o_ref):
    num = n0_ref[...] + n1_ref[...]
    den = d0_ref[...] + d1_ref[...]
    recip = 1.0 / (den + 1e-16)
    denx = jnp.dot(recip, s_ref[...], preferred_element_type=jnp.float32)
    o_ref[...] = jnp.maximum(num * denx + b_ref[...][0:1, :], 0.0)


def _make_sc_kernel(npad, epad, H, C):
    HC = H * C
    ept = epad // NTILES
    nch = ept // CB
    rps = npad // NSUB

    mesh = plsc.VectorSubcoreMesh(
        core_axis_name="c", subcore_axis_name="s",
        num_cores=NCORES, num_subcores=NSUB)

    @functools.partial(
        pl.kernel,
        out_type=[
            jax.ShapeDtypeStruct((npad, HC), jnp.float32),
            jax.ShapeDtypeStruct((npad, HC), jnp.float32),
            jax.ShapeDtypeStruct((npad // 8, 128), jnp.float32),
            jax.ShapeDtypeStruct((npad // 8, 128), jnp.float32),
        ],
        mesh=mesh,
        compiler_params=pltpu.CompilerParams(needs_layout_passes=False),
        scratch_types=[
            pltpu.VMEM((HC,), jnp.float32),
            pltpu.VMEM((CB,), jnp.int32),
            pltpu.VMEM((CB,), jnp.int32),
            pltpu.VMEM((CB,), jnp.int32),
            pltpu.VMEM((CB, HC), jnp.float32),
            pltpu.VMEM((CB, HC), jnp.float32),
            pltpu.VMEM_SHARED((npad, HC), jnp.float32),
            pltpu.VMEM_SHARED((npad // 8, 128), jnp.float32),
            pltpu.SemaphoreType.DMA,
        ],
    )
    def sc_kernel(xl_h, xr_h, src_h, dst_h, att_h,
                  num0_o, num1_o, den0_o, den1_o,
                  att_v, src_v, dst_v, dstpk_v, xlr, xrr, num_sh, den_sh,
                  sem):
        cid = lax.axis_index("c")
        sid = lax.axis_index("s")
        r0 = sid * rps
        pr0 = sid * (rps // 8)
        iota = lax.iota(jnp.int32, 16)
        zero16 = jnp.zeros((16,), jnp.float32)

        pltpu.sync_copy(att_h, att_v)
        for r in range(8):
            for c8 in range(HC // 16):
                xlr[r, pl.ds(c8 * 16, 16)] = zero16

        def zinit(t, carry):
            pltpu.sync_copy(xlr.at[pl.ds(0, 8)], num_sh.at[pl.ds(r0 + t * 8, 8)])
            return carry
        lax.fori_loop(0, rps // 8, zinit, 0)

        def zinit2(t, carry):
            pltpu.sync_copy(xlr.at[pl.ds(0, 8)], den_sh.at[pl.ds(pr0 + t * 8, 8)])
            return carry
        lax.fori_loop(0, rps // 64, zinit2, 0)

        plsc.subcore_barrier()
        tid = cid * NSUB + sid
        ebase0 = tid * ept
        nv = HC // 16
        attv = [att_v[pl.ds(k * 16, 16)] for k in range(nv)]

        def chunk(gc, attv):
            eb = ebase0 + gc * CB
            pltpu.sync_copy(src_h.at[pl.ds(eb, CB)], src_v)
            pltpu.sync_copy(dst_h.at[pl.ds(eb, CB)], dst_v)
            pltpu.async_copy(xl_h.at[src_v], xlr, sem).wait()
            pltpu.async_copy(xr_h.at[dst_v], xrr, sem).wait()
            for g16 in range(CB // 16):
                dv = dst_v[pl.ds(g16 * 16, 16)]
                dstpk_v[pl.ds(g16 * 16, 16)] = lax.shift_right_logical(dv, 3)

            kph = C // 16
            def edge(r, carry):
                rv = jnp.full((16,), r, jnp.int32)
                roww = zero16
                for h in range(H):
                    acc = zero16
                    xsl = []
                    for k in range(kph):
                        sl = pl.ds((h * kph + k) * 16, 16)
                        xv = xlr[r, sl]
                        xsl.append(xv)
                        v = xv + xrr[r, sl]
                        v = jnp.maximum(v, NEG_SLOPE * v)
                        acc = acc + attv[h * kph + k] * v
                    w = jnp.exp(jnp.full((16,), jnp.sum(acc)))
                    for k in range(kph):
                        sl = pl.ds((h * kph + k) * 16, 16)
                        xlr[r, sl] = xsl[k] * w
                    roww = jnp.where(iota == h, w, roww)
                dsplat = plsc.load_gather(dst_v, [rv])
                for s8 in range(nv):
                    xrr[r, pl.ds(s8 * 16, 16)] = zero16
                colv = (dsplat & 7) * 16 + iota
                plsc.store_scatter(xrr, [rv, colv], roww)
                return carry
            plsc.parallel_loop(0, CB, 1, unroll=8, carry=jnp.int32(0))(edge)

            pltpu.sync_copy(xlr, num_sh.at[dst_v], add=True)
            pltpu.sync_copy(xrr, den_sh.at[dstpk_v], add=True)
            return attv

        lax.fori_loop(0, nch, chunk, attv)
        plsc.subcore_barrier()

        def copy_out(num_o, den_o):
            def cp(t, carry):
                rr = r0 + t * CB
                prr = pr0 + t * 8
                pltpu.sync_copy(num_sh.at[pl.ds(rr, CB)], xlr)
                pltpu.sync_copy(xlr, num_o.at[pl.ds(rr, CB)])
                pltpu.sync_copy(den_sh.at[pl.ds(prr, 8)], xrr.at[pl.ds(0, 8)])
                pltpu.sync_copy(xrr.at[pl.ds(0, 8)], den_o.at[pl.ds(prr, 8)])
                return carry
            lax.fori_loop(0, rps // CB, cp, 0)

        @pl.when(cid == 0)
        def _():
            copy_out(num0_o, den0_o)

        @pl.when(cid == 1)
        def _():
            copy_out(num1_o, den1_o)

    return sc_kernel


def kernel(x, edge_index, W_l, W_r, att, bias):
    n, in_dim = x.shape
    H, C = att.shape
    HC = H * C
    e = edge_index.shape[1]
    etot = e + n
    npad = ((n + 1 + 1023) // 1024) * 1024
    epad = ((etot + CB * NTILES - 1) // (CB * NTILES)) * (CB * NTILES)

    x_pad = jnp.zeros((npad, in_dim), jnp.float32).at[:n].set(x)
    loop = jnp.arange(n, dtype=jnp.int32)
    padi = jnp.full((epad - etot,), n, dtype=jnp.int32)
    src_all = jnp.concatenate([edge_index[0], loop, padi])
    dst_all = jnp.concatenate([edge_index[1], loop, padi])
    att_flat = att.reshape(HC)

    nblk = 8
    brows = npad // nblk
    xl, xr = pl.pallas_call(
        _proj_body,
        grid=(nblk,),
        in_specs=[
            pl.BlockSpec((brows, in_dim), lambda i: (i, 0)),
            pl.BlockSpec((in_dim, HC), lambda i: (0, 0)),
            pl.BlockSpec((in_dim, HC), lambda i: (0, 0)),
        ],
        out_specs=[
            pl.BlockSpec((brows, HC), lambda i: (i, 0)),
            pl.BlockSpec((brows, HC), lambda i: (i, 0)),
        ],
        out_shape=[
            jax.ShapeDtypeStruct((npad, HC), jnp.float32),
            jax.ShapeDtypeStruct((npad, HC), jnp.float32),
        ],
    )(x_pad, W_l, W_r)

    sc_kernel = _make_sc_kernel(npad, epad, H, C)
    num0, num1, den0, den1 = sc_kernel(
        xl, xr, src_all, dst_all, att_flat)
    den0 = den0.reshape(npad, 16)
    den1 = den1.reshape(npad, 16)

    sel = jnp.zeros((16, HC), jnp.float32).at[:H].set(
        jnp.repeat(jnp.eye(H, dtype=jnp.float32), C, axis=1))
    bias_b = jnp.broadcast_to(bias.reshape(1, HC), (8, HC))
    out = pl.pallas_call(
        _fin_body,
        grid=(nblk,),
        in_specs=[
            pl.BlockSpec((brows, HC), lambda i: (i, 0)),
            pl.BlockSpec((brows, HC), lambda i: (i, 0)),
            pl.BlockSpec((brows, 16), lambda i: (i, 0)),
            pl.BlockSpec((brows, 16), lambda i: (i, 0)),
            pl.BlockSpec((8, HC), lambda i: (0, 0)),
            pl.BlockSpec((16, HC), lambda i: (0, 0)),
        ],
        out_specs=pl.BlockSpec((brows, HC), lambda i: (i, 0)),
        out_shape=jax.ShapeDtypeStruct((npad, HC), jnp.float32),
    )(num0, num1, den0, den1, bias_b, sel)
    return out[:n]


# speedup vs baseline: 50.6900x; 1.2057x over previous
import functools

import jax
import jax.numpy as jnp
from jax import lax
from jax.experimental import pallas as pl
from jax.experimental.pallas import tpu as pltpu
from jax.experimental.pallas import tpu_sc as plsc

NEG_SLOPE = 0.2
CB = 64
NCORES = 2
NSUB = 16
NTILES = NCORES * NSUB


def _proj_body(x_ref, wl_ref, wr_ref, xl_ref, xr_ref):
    xv = x_ref[...]
    xl_ref[...] = jnp.dot(xv, wl_ref[...], preferred_element_type=jnp.float32)
    xr_ref[...] = jnp.dot(xv, wr_ref[...], preferred_element_type=jnp.float32)


def _fin_body(n0_ref, n1_ref, d0_ref, d1_ref, b_ref, s_ref, o_ref):
    num = n0_ref[...] + n1_ref[...]
    den = d0_ref[...] + d1_ref[...]
    recip = 1.0 / (den + 1e-16)
    denx = jnp.dot(recip, s_ref[...], preferred_element_type=jnp.float32)
    o_ref[...] = jnp.maximum(num * denx + b_ref[...][0:1, :], 0.0)


def _make_sc_kernel(npad, epad, H, C):
    HC = H * C
    ept = epad // NTILES
    nch = ept // CB
    rps = npad // NSUB

    mesh = plsc.VectorSubcoreMesh(
        core_axis_name="c", subcore_axis_name="s",
        num_cores=NCORES, num_subcores=NSUB)

    @functools.partial(
        pl.kernel,
        out_type=[
            jax.ShapeDtypeStruct((npad, HC), jnp.float32),
            jax.ShapeDtypeStruct((npad, HC), jnp.float32),
            jax.ShapeDtypeStruct((npad // 8, 128), jnp.float32),
            jax.ShapeDtypeStruct((npad // 8, 128), jnp.float32),
        ],
        mesh=mesh,
        compiler_params=pltpu.CompilerParams(needs_layout_passes=False),
        scratch_types=[
            pltpu.VMEM((HC,), jnp.float32),
            pltpu.VMEM((CB,), jnp.int32),
            pltpu.VMEM((CB,), jnp.int32),
            pltpu.VMEM((CB,), jnp.int32),
            pltpu.VMEM((CB, HC), jnp.float32),
            pltpu.VMEM((CB, HC), jnp.float32),
            pltpu.VMEM_SHARED((npad, HC), jnp.float32),
            pltpu.VMEM_SHARED((npad // 8, 128), jnp.float32),
            pltpu.SemaphoreType.DMA,
        ],
    )
    def sc_kernel(xl_h, xr_h, src_h, dst_h, att_h,
                  num0_o, num1_o, den0_o, den1_o,
                  att_v, src_v, dst_v, dstpk_v, xlr, xrr, num_sh, den_sh,
                  sem):
        cid = lax.axis_index("c")
        sid = lax.axis_index("s")
        r0 = sid * rps
        pr0 = sid * (rps // 8)
        iota = lax.iota(jnp.int32, 16)
        zero16 = jnp.zeros((16,), jnp.float32)

        pltpu.sync_copy(att_h, att_v)
        for r in range(8):
            for c8 in range(HC // 16):
                xlr[r, pl.ds(c8 * 16, 16)] = zero16

        def zinit(t, carry):
            pltpu.sync_copy(xlr.at[pl.ds(0, 8)], num_sh.at[pl.ds(r0 + t * 8, 8)])
            return carry
        lax.fori_loop(0, rps // 8, zinit, 0)

        def zinit2(t, carry):
            pltpu.sync_copy(xlr.at[pl.ds(0, 8)], den_sh.at[pl.ds(pr0 + t * 8, 8)])
            return carry
        lax.fori_loop(0, rps // 64, zinit2, 0)

        plsc.subcore_barrier()
        tid = cid * NSUB + sid
        ebase0 = tid * ept
        nv = HC // 16
        attv = [att_v[pl.ds(k * 16, 16)] for k in range(nv)]

        def chunk(gc, attv):
            eb = ebase0 + gc * CB
            pltpu.sync_copy(src_h.at[pl.ds(eb, CB)], src_v)
            pltpu.sync_copy(dst_h.at[pl.ds(eb, CB)], dst_v)
            cp1 = pltpu.async_copy(xl_h.at[src_v], xlr, sem)
            cp2 = pltpu.async_copy(xr_h.at[dst_v], xrr, sem)
            cp1.wait()
            cp2.wait()
            for g16 in range(CB // 16):
                dv = dst_v[pl.ds(g16 * 16, 16)]
                dstpk_v[pl.ds(g16 * 16, 16)] = lax.shift_right_logical(dv, 3)

            kph = C // 16
            def edge(r, carry):
                rv = jnp.full((16,), r, jnp.int32)
                roww = zero16
                for h in range(H):
                    acc = zero16
                    xsl = []
                    for k in range(kph):
                        sl = pl.ds((h * kph + k) * 16, 16)
                        xv = xlr[r, sl]
                        xsl.append(xv)
                        v = xv + xrr[r, sl]
                        v = jnp.maximum(v, NEG_SLOPE * v)
                        acc = acc + attv[h * kph + k] * v
                    w = jnp.exp(jnp.full((16,), jnp.sum(acc)))
                    for k in range(kph):
                        sl = pl.ds((h * kph + k) * 16, 16)
                        xlr[r, sl] = xsl[k] * w
                    roww = jnp.where(iota == h, w, roww)
                dsplat = plsc.load_gather(dst_v, [rv])
                for s8 in range(nv):
                    xrr[r, pl.ds(s8 * 16, 16)] = zero16
                colv = (dsplat & 7) * 16 + iota
                plsc.store_scatter(xrr, [rv, colv], roww)
                return carry
            plsc.parallel_loop(0, CB, 1, unroll=8, carry=jnp.int32(0))(edge)

            pltpu.sync_copy(xlr, num_sh.at[dst_v], add=True)
            pltpu.sync_copy(xrr, den_sh.at[dstpk_v], add=True)
            return attv

        lax.fori_loop(0, nch, chunk, attv)
        plsc.subcore_barrier()

        def copy_out(num_o, den_o):
            def cp(t, carry):
                rr = r0 + t * CB
                prr = pr0 + t * 8
                pltpu.sync_copy(num_sh.at[pl.ds(rr, CB)], xlr)
                pltpu.sync_copy(xlr, num_o.at[pl.ds(rr, CB)])
                pltpu.sync_copy(den_sh.at[pl.ds(prr, 8)], xrr.at[pl.ds(0, 8)])
                pltpu.sync_copy(xrr.at[pl.ds(0, 8)], den_o.at[pl.ds(prr, 8)])
                return carry
            lax.fori_loop(0, rps // CB, cp, 0)

        @pl.when(cid == 0)
        def _():
            copy_out(num0_o, den0_o)

        @pl.when(cid == 1)
        def _():
            copy_out(num1_o, den1_o)

    return sc_kernel


def kernel(x, edge_index, W_l, W_r, att, bias):
    n, in_dim = x.shape
    H, C = att.shape
    HC = H * C
    e = edge_index.shape[1]
    etot = e + n
    npad = ((n + 1 + 1023) // 1024) * 1024
    epad = ((etot + CB * NTILES - 1) // (CB * NTILES)) * (CB * NTILES)

    x_pad = jnp.zeros((npad, in_dim), jnp.float32).at[:n].set(x)
    loop = jnp.arange(n, dtype=jnp.int32)
    padi = jnp.full((epad - etot,), n, dtype=jnp.int32)
    src_all = jnp.concatenate([edge_index[0], loop, padi])
    dst_all = jnp.concatenate([edge_index[1], loop, padi])
    att_flat = att.reshape(HC)

    nblk = 8
    brows = npad // nblk
    xl, xr = pl.pallas_call(
        _proj_body,
        grid=(nblk,),
        in_specs=[
            pl.BlockSpec((brows, in_dim), lambda i: (i, 0)),
            pl.BlockSpec((in_dim, HC), lambda i: (0, 0)),
            pl.BlockSpec((in_dim, HC), lambda i: (0, 0)),
        ],
        out_specs=[
            pl.BlockSpec((brows, HC), lambda i: (i, 0)),
            pl.BlockSpec((brows, HC), lambda i: (i, 0)),
        ],
        out_shape=[
            jax.ShapeDtypeStruct((npad, HC), jnp.float32),
            jax.ShapeDtypeStruct((npad, HC), jnp.float32),
        ],
    )(x_pad, W_l, W_r)

    sc_kernel = _make_sc_kernel(npad, epad, H, C)
    num0, num1, den0, den1 = sc_kernel(
        xl, xr, src_all, dst_all, att_flat)
    den0 = den0.reshape(npad, 16)
    den1 = den1.reshape(npad, 16)

    sel = jnp.zeros((16, HC), jnp.float32).at[:H].set(
        jnp.repeat(jnp.eye(H, dtype=jnp.float32), C, axis=1))
    bias_b = jnp.broadcast_to(bias.reshape(1, HC), (8, HC))
    out = pl.pallas_call(
        _fin_body,
        grid=(nblk,),
        in_specs=[
            pl.BlockSpec((brows, HC), lambda i: (i, 0)),
            pl.BlockSpec((brows, HC), lambda i: (i, 0)),
            pl.BlockSpec((brows, 16), lambda i: (i, 0)),
            pl.BlockSpec((brows, 16), lambda i: (i, 0)),
            pl.BlockSpec((8, HC), lambda i: (0, 0)),
            pl.BlockSpec((16, HC), lambda i: (0, 0)),
        ],
        out_specs=pl.BlockSpec((brows, HC), lambda i: (i, 0)),
        out_shape=jax.ShapeDtypeStruct((npad, HC), jnp.float32),
    )(num0, num1, den0, den1, bias_b, sel)
    return out[:n]


# speedup vs baseline: 59.2276x; 1.1684x over previous
import functools

import jax
import jax.numpy as jnp
from jax import lax
from jax.experimental import pallas as pl
from jax.experimental.pallas import tpu as pltpu
from jax.experimental.pallas import tpu_sc as plsc

NEG_SLOPE = 0.2
CB = 64
NCORES = 2
NSUB = 16
NTILES = NCORES * NSUB


def _proj_body(x_ref, wl_ref, wr_ref, xl_ref, xr_ref):
    xv = x_ref[...]
    xl_ref[...] = jnp.dot(xv, wl_ref[...], preferred_element_type=jnp.float32)
    xr_ref[...] = jnp.dot(xv, wr_ref[...], preferred_element_type=jnp.float32)


def _fin_body(n0_ref, n1_ref, d0_ref, d1_ref, b_ref, s_ref, o_ref):
    num = n0_ref[...] + n1_ref[...]
    den = d0_ref[...] + d1_ref[...]
    recip = 1.0 / (den + 1e-16)
    denx = jnp.dot(recip, s_ref[...], preferred_element_type=jnp.float32)
    o_ref[...] = jnp.maximum(num * denx + b_ref[...][0:1, :], 0.0)


def _make_sc_kernel(npad, epad, H, C):
    HC = H * C
    ept = epad // NTILES
    HB = CB // 2
    nh = ept // HB
    rps = npad // NSUB

    mesh = plsc.VectorSubcoreMesh(
        core_axis_name="c", subcore_axis_name="s",
        num_cores=NCORES, num_subcores=NSUB)

    @functools.partial(
        pl.kernel,
        out_type=[
            jax.ShapeDtypeStruct((npad, HC), jnp.float32),
            jax.ShapeDtypeStruct((npad, HC), jnp.float32),
            jax.ShapeDtypeStruct((npad // 8, 128), jnp.float32),
            jax.ShapeDtypeStruct((npad // 8, 128), jnp.float32),
        ],
        mesh=mesh,
        compiler_params=pltpu.CompilerParams(needs_layout_passes=False),
        scratch_types=[
            pltpu.VMEM((HC,), jnp.float32),
            pltpu.VMEM((HB,), jnp.int32),
            pltpu.VMEM((HB,), jnp.int32),
            pltpu.VMEM((HB,), jnp.int32),
            pltpu.VMEM((HB,), jnp.int32),
            pltpu.VMEM((HB,), jnp.int32),
            pltpu.VMEM((HB,), jnp.int32),
            pltpu.VMEM((HB, HC), jnp.float32),
            pltpu.VMEM((HB, HC), jnp.float32),
            pltpu.VMEM((HB, HC), jnp.float32),
            pltpu.VMEM((HB, HC), jnp.float32),
            pltpu.VMEM_SHARED((npad, HC), jnp.float32),
            pltpu.VMEM_SHARED((npad // 8, 128), jnp.float32),
            pltpu.SemaphoreType.DMA,
            pltpu.SemaphoreType.DMA,
        ],
    )
    def sc_kernel(xl_h, xr_h, src_h, dst_h, att_h,
                  num0_o, num1_o, den0_o, den1_o,
                  att_v, src_a, dst_a, pk_a, src_b, dst_b, pk_b,
                  xla, xra, xlb, xrb, num_sh, den_sh, sema, semb):
        cid = lax.axis_index("c")
        sid = lax.axis_index("s")
        r0 = sid * rps
        pr0 = sid * (rps // 8)
        iota = lax.iota(jnp.int32, 16)
        zero16 = jnp.zeros((16,), jnp.float32)
        nv = HC // 16

        pltpu.sync_copy(att_h, att_v)
        for r in range(8):
            for c8 in range(nv):
                xla[r, pl.ds(c8 * 16, 16)] = zero16

        def zinit(t, carry):
            pltpu.sync_copy(xla.at[pl.ds(0, 8)], num_sh.at[pl.ds(r0 + t * 8, 8)])
            return carry
        lax.fori_loop(0, rps // 8, zinit, 0)

        def zinit2(t, carry):
            pltpu.sync_copy(xla.at[pl.ds(0, 8)], den_sh.at[pl.ds(pr0 + t * 8, 8)])
            return carry
        lax.fori_loop(0, rps // 64, zinit2, 0)

        plsc.subcore_barrier()
        tid = cid * NSUB + sid
        ebase0 = tid * ept
        attv = [att_v[pl.ds(k * 16, 16)] for k in range(nv)]
        kph = C // 16

        def issue(idx, sv, dv, xlr, xrr, sem):
            eb = ebase0 + idx * HB
            pltpu.sync_copy(src_h.at[pl.ds(eb, HB)], sv)
            pltpu.sync_copy(dst_h.at[pl.ds(eb, HB)], dv)
            pltpu.async_copy(xl_h.at[sv], xlr, sem)
            pltpu.async_copy(xr_h.at[dv], xrr, sem)

        def drain(xlr, xrr, sem):
            pltpu.make_async_copy(xl_h.at[pl.ds(0, HB)], xlr, sem).wait()
            pltpu.make_async_copy(xl_h.at[pl.ds(0, HB)], xrr, sem).wait()

        def process(sv, dv, pkv, xlr, xrr):
            for g16 in range(HB // 16):
                dvv = dv[pl.ds(g16 * 16, 16)]
                pkv[pl.ds(g16 * 16, 16)] = lax.shift_right_logical(dvv, 3)

            def edge(r, carry):
                rv = jnp.full((16,), r, jnp.int32)
                roww = zero16
                for h in range(H):
                    acc = zero16
                    xsl = []
                    for k in range(kph):
                        sl = pl.ds((h * kph + k) * 16, 16)
                        xv = xlr[r, sl]
                        xsl.append(xv)
                        v = xv + xrr[r, sl]
                        v = jnp.maximum(v, NEG_SLOPE * v)
                        acc = acc + attv[h * kph + k] * v
                    w = jnp.exp(jnp.full((16,), jnp.sum(acc)))
                    for k in range(kph):
                        sl = pl.ds((h * kph + k) * 16, 16)
                        xlr[r, sl] = xsl[k] * w
                    roww = jnp.where(iota == h, w, roww)
                dsplat = plsc.load_gather(dv, [rv])
                for s8 in range(nv):
                    xrr[r, pl.ds(s8 * 16, 16)] = zero16
                colv = (dsplat & 7) * 16 + iota
                plsc.store_scatter(xrr, [rv, colv], roww)
                return carry
            plsc.parallel_loop(0, HB, 1, unroll=8, carry=jnp.int32(0))(edge)

            pltpu.sync_copy(xlr, num_sh.at[dv], add=True)
            pltpu.sync_copy(xrr, den_sh.at[pkv], add=True)

        issue(0, src_a, dst_a, xla, xra, sema)

        def chunk(gc, carry):
            i0 = gc * 2
            issue(i0 + 1, src_b, dst_b, xlb, xrb, semb)
            drain(xla, xra, sema)
            process(src_a, dst_a, pk_a, xla, xra)

            @pl.when(i0 + 2 < nh)
            def _():
                issue(i0 + 2, src_a, dst_a, xla, xra, sema)
            drain(xlb, xrb, semb)
            process(src_b, dst_b, pk_b, xlb, xrb)
            return carry

        lax.fori_loop(0, nh // 2, chunk, 0)
        plsc.subcore_barrier()

        def copy_out(num_o, den_o):
            def cp(t, carry):
                rr = r0 + t * HB
                pltpu.sync_copy(num_sh.at[pl.ds(rr, HB)], xla)
                pltpu.sync_copy(xla, num_o.at[pl.ds(rr, HB)])
                return carry
            lax.fori_loop(0, rps // HB, cp, 0)

            def cpd(t, carry):
                prr = pr0 + t * 8
                pltpu.sync_copy(den_sh.at[pl.ds(prr, 8)], xra.at[pl.ds(0, 8)])
                pltpu.sync_copy(xra.at[pl.ds(0, 8)], den_o.at[pl.ds(prr, 8)])
                return carry
            lax.fori_loop(0, rps // 64, cpd, 0)

        @pl.when(cid == 0)
        def _():
            copy_out(num0_o, den0_o)

        @pl.when(cid == 1)
        def _():
            copy_out(num1_o, den1_o)

    return sc_kernel


def kernel(x, edge_index, W_l, W_r, att, bias):
    n, in_dim = x.shape
    H, C = att.shape
    HC = H * C
    e = edge_index.shape[1]
    etot = e + n
    npad = ((n + 1 + 1023) // 1024) * 1024
    epad = ((etot + CB * NTILES - 1) // (CB * NTILES)) * (CB * NTILES)

    x_pad = jnp.zeros((npad, in_dim), jnp.float32).at[:n].set(x)
    loop = jnp.arange(n, dtype=jnp.int32)
    padi = jnp.full((epad - etot,), n, dtype=jnp.int32)
    src_all = jnp.concatenate([edge_index[0], loop, padi])
    dst_all = jnp.concatenate([edge_index[1], loop, padi])
    att_flat = att.reshape(HC)

    nblk = 8
    brows = npad // nblk
    xl, xr = pl.pallas_call(
        _proj_body,
        grid=(nblk,),
        in_specs=[
            pl.BlockSpec((brows, in_dim), lambda i: (i, 0)),
            pl.BlockSpec((in_dim, HC), lambda i: (0, 0)),
            pl.BlockSpec((in_dim, HC), lambda i: (0, 0)),
        ],
        out_specs=[
            pl.BlockSpec((brows, HC), lambda i: (i, 0)),
            pl.BlockSpec((brows, HC), lambda i: (i, 0)),
        ],
        out_shape=[
            jax.ShapeDtypeStruct((npad, HC), jnp.float32),
            jax.ShapeDtypeStruct((npad, HC), jnp.float32),
        ],
    )(x_pad, W_l, W_r)

    sc_kernel = _make_sc_kernel(npad, epad, H, C)
    num0, num1, den0, den1 = sc_kernel(
        xl, xr, src_all, dst_all, att_flat)
    den0 = den0.reshape(npad, 16)
    den1 = den1.reshape(npad, 16)

    sel = jnp.zeros((16, HC), jnp.float32).at[:H].set(
        jnp.repeat(jnp.eye(H, dtype=jnp.float32), C, axis=1))
    bias_b = jnp.broadcast_to(bias.reshape(1, HC), (8, HC))
    out = pl.pallas_call(
        _fin_body,
        grid=(nblk,),
        in_specs=[
            pl.BlockSpec((brows, HC), lambda i: (i, 0)),
            pl.BlockSpec((brows, HC), lambda i: (i, 0)),
            pl.BlockSpec((brows, 16), lambda i: (i, 0)),
            pl.BlockSpec((brows, 16), lambda i: (i, 0)),
            pl.BlockSpec((8, HC), lambda i: (0, 0)),
            pl.BlockSpec((16, HC), lambda i: (0, 0)),
        ],
        out_specs=pl.BlockSpec((brows, HC), lambda i: (i, 0)),
        out_shape=jax.ShapeDtypeStruct((npad, HC), jnp.float32),
    )(num0, num1, den0, den1, bias_b, sel)
    return out[:n]


# speedup vs baseline: 60.6566x; 1.0241x over previous
import functools

import jax
import jax.numpy as jnp
from jax import lax
from jax.experimental import pallas as pl
from jax.experimental.pallas import tpu as pltpu
from jax.experimental.pallas import tpu_sc as plsc

NEG_SLOPE = 0.2
CB = 64
NCORES = 2
NSUB = 16
NTILES = NCORES * NSUB


def _proj_body(x_ref, wl_ref, wr_ref, xl_ref, xr_ref):
    xv = x_ref[...]
    xl_ref[...] = jnp.dot(xv, wl_ref[...], preferred_element_type=jnp.float32)
    xr_ref[...] = jnp.dot(xv, wr_ref[...], preferred_element_type=jnp.float32)


def _fin_body(n0_ref, n1_ref, d0_ref, d1_ref, b_ref, s_ref, o_ref):
    num = n0_ref[...] + n1_ref[...]
    den = d0_ref[...] + d1_ref[...]
    recip = 1.0 / (den + 1e-16)
    denx = jnp.dot(recip, s_ref[...], preferred_element_type=jnp.float32)
    o_ref[...] = jnp.maximum(num * denx + b_ref[...][0:1, :], 0.0)


def _make_sc_kernel(npad, epad, H, C):
    HC = H * C
    ept = epad // NTILES
    HB = CB // 2
    nh = ept // HB
    rps = npad // NSUB

    mesh = plsc.VectorSubcoreMesh(
        core_axis_name="c", subcore_axis_name="s",
        num_cores=NCORES, num_subcores=NSUB)

    @functools.partial(
        pl.kernel,
        out_type=[
            jax.ShapeDtypeStruct((npad, HC), jnp.float32),
            jax.ShapeDtypeStruct((npad, HC), jnp.float32),
            jax.ShapeDtypeStruct((npad // 8, 128), jnp.float32),
            jax.ShapeDtypeStruct((npad // 8, 128), jnp.float32),
        ],
        mesh=mesh,
        compiler_params=pltpu.CompilerParams(needs_layout_passes=False),
        scratch_types=[
            pltpu.VMEM((HC,), jnp.float32),
            pltpu.VMEM((HB,), jnp.int32),
            pltpu.VMEM((HB,), jnp.int32),
            pltpu.VMEM((HB,), jnp.int32),
            pltpu.VMEM((HB,), jnp.int32),
            pltpu.VMEM((HB,), jnp.int32),
            pltpu.VMEM((HB,), jnp.int32),
            pltpu.VMEM((HB, HC), jnp.float32),
            pltpu.VMEM((HB, HC), jnp.float32),
            pltpu.VMEM((HB, HC), jnp.float32),
            pltpu.VMEM((HB, HC), jnp.float32),
            pltpu.VMEM_SHARED((npad, HC), jnp.float32),
            pltpu.VMEM_SHARED((npad // 8, 128), jnp.float32),
            pltpu.SemaphoreType.DMA,
            pltpu.SemaphoreType.DMA,
            pltpu.SemaphoreType.DMA,
        ],
    )
    def sc_kernel(xl_h, xr_h, src_h, dst_h, att_h,
                  num0_o, num1_o, den0_o, den1_o,
                  att_v, src_a, dst_a, pk_a, src_b, dst_b, pk_b,
                  xla, xra, xlb, xrb, num_sh, den_sh, sema, semb, semc):
        cid = lax.axis_index("c")
        sid = lax.axis_index("s")
        r0 = sid * rps
        pr0 = sid * (rps // 8)
        iota = lax.iota(jnp.int32, 16)
        zero16 = jnp.zeros((16,), jnp.float32)
        nv = HC // 16

        pltpu.sync_copy(att_h, att_v)
        for r in range(8):
            for c8 in range(nv):
                xla[r, pl.ds(c8 * 16, 16)] = zero16

        def zinit(t, carry):
            pltpu.sync_copy(xla.at[pl.ds(0, 8)], num_sh.at[pl.ds(r0 + t * 8, 8)])
            return carry
        lax.fori_loop(0, rps // 8, zinit, 0)

        def zinit2(t, carry):
            pltpu.sync_copy(xla.at[pl.ds(0, 8)], den_sh.at[pl.ds(pr0 + t * 8, 8)])
            return carry
        lax.fori_loop(0, rps // 64, zinit2, 0)

        plsc.subcore_barrier()
        tid = cid * NSUB + sid
        ebase0 = tid * ept
        attv = [att_v[pl.ds(k * 16, 16)] for k in range(nv)]
        kph = C // 16

        def issue(idx, sv, dv, xlr, xrr, sem):
            eb = ebase0 + idx * HB
            pltpu.sync_copy(src_h.at[pl.ds(eb, HB)], sv)
            pltpu.sync_copy(dst_h.at[pl.ds(eb, HB)], dv)
            pltpu.async_copy(xl_h.at[sv], xlr, sem)
            pltpu.async_copy(xr_h.at[dv], xrr, sem)

        def drain(xlr, xrr, sem):
            pltpu.make_async_copy(xl_h.at[pl.ds(0, HB)], xlr, sem).wait()
            pltpu.make_async_copy(xl_h.at[pl.ds(0, HB)], xrr, sem).wait()

        def process(sv, dv, pkv, xlr, xrr):
            for g16 in range(HB // 16):
                dvv = dv[pl.ds(g16 * 16, 16)]
                pkv[pl.ds(g16 * 16, 16)] = lax.shift_right_logical(dvv, 3)

            def edge(r, carry):
                rv = jnp.full((16,), r, jnp.int32)
                roww = zero16
                for h in range(H):
                    acc = zero16
                    xsl = []
                    for k in range(kph):
                        sl = pl.ds((h * kph + k) * 16, 16)
                        xv = xlr[r, sl]
                        xsl.append(xv)
                        v = xv + xrr[r, sl]
                        v = jnp.maximum(v, NEG_SLOPE * v)
                        acc = acc + attv[h * kph + k] * v
                    w = jnp.exp(jnp.full((16,), jnp.sum(acc)))
                    for k in range(kph):
                        sl = pl.ds((h * kph + k) * 16, 16)
                        xlr[r, sl] = xsl[k] * w
                    roww = jnp.where(iota == h, w, roww)
                dsplat = plsc.load_gather(dv, [rv])
                for s8 in range(nv):
                    xrr[r, pl.ds(s8 * 16, 16)] = zero16
                colv = (dsplat & 7) * 16 + iota
                plsc.store_scatter(xrr, [rv, colv], roww)
                return carry
            plsc.parallel_loop(0, HB, 1, unroll=8, carry=jnp.int32(0))(edge)

            s1 = pltpu.async_copy(xlr, num_sh.at[dv], semc, add=True)
            s2 = pltpu.async_copy(xrr, den_sh.at[pkv], semc, add=True)
            s1.wait()
            s2.wait()

        issue(0, src_a, dst_a, xla, xra, sema)

        def chunk(gc, carry):
            i0 = gc * 2
            issue(i0 + 1, src_b, dst_b, xlb, xrb, semb)
            drain(xla, xra, sema)
            process(src_a, dst_a, pk_a, xla, xra)

            @pl.when(i0 + 2 < nh)
            def _():
                issue(i0 + 2, src_a, dst_a, xla, xra, sema)
            drain(xlb, xrb, semb)
            process(src_b, dst_b, pk_b, xlb, xrb)
            return carry

        lax.fori_loop(0, nh // 2, chunk, 0)
        plsc.subcore_barrier()

        def copy_out(num_o, den_o):
            def cp(t, carry):
                rr = r0 + t * HB
                pltpu.sync_copy(num_sh.at[pl.ds(rr, HB)], xla)
                pltpu.sync_copy(xla, num_o.at[pl.ds(rr, HB)])
                return carry
            lax.fori_loop(0, rps // HB, cp, 0)

            def cpd(t, carry):
                prr = pr0 + t * 8
                pltpu.sync_copy(den_sh.at[pl.ds(prr, 8)], xra.at[pl.ds(0, 8)])
                pltpu.sync_copy(xra.at[pl.ds(0, 8)], den_o.at[pl.ds(prr, 8)])
                return carry
            lax.fori_loop(0, rps // 64, cpd, 0)

        @pl.when(cid == 0)
        def _():
            copy_out(num0_o, den0_o)

        @pl.when(cid == 1)
        def _():
            copy_out(num1_o, den1_o)

    return sc_kernel


def kernel(x, edge_index, W_l, W_r, att, bias):
    n, in_dim = x.shape
    H, C = att.shape
    HC = H * C
    e = edge_index.shape[1]
    etot = e + n
    npad = ((n + 1 + 1023) // 1024) * 1024
    epad = ((etot + CB * NTILES - 1) // (CB * NTILES)) * (CB * NTILES)

    x_pad = jnp.zeros((npad, in_dim), jnp.float32).at[:n].set(x)
    loop = jnp.arange(n, dtype=jnp.int32)
    padi = jnp.full((epad - etot,), n, dtype=jnp.int32)
    src_all = jnp.concatenate([edge_index[0], loop, padi])
    dst_all = jnp.concatenate([edge_index[1], loop, padi])
    att_flat = att.reshape(HC)

    nblk = 8
    brows = npad // nblk
    xl, xr = pl.pallas_call(
        _proj_body,
        grid=(nblk,),
        in_specs=[
            pl.BlockSpec((brows, in_dim), lambda i: (i, 0)),
            pl.BlockSpec((in_dim, HC), lambda i: (0, 0)),
            pl.BlockSpec((in_dim, HC), lambda i: (0, 0)),
        ],
        out_specs=[
            pl.BlockSpec((brows, HC), lambda i: (i, 0)),
            pl.BlockSpec((brows, HC), lambda i: (i, 0)),
        ],
        out_shape=[
            jax.ShapeDtypeStruct((npad, HC), jnp.float32),
            jax.ShapeDtypeStruct((npad, HC), jnp.float32),
        ],
    )(x_pad, W_l, W_r)

    sc_kernel = _make_sc_kernel(npad, epad, H, C)
    num0, num1, den0, den1 = sc_kernel(
        xl, xr, src_all, dst_all, att_flat)
    den0 = den0.reshape(npad, 16)
    den1 = den1.reshape(npad, 16)

    sel = jnp.zeros((16, HC), jnp.float32).at[:H].set(
        jnp.repeat(jnp.eye(H, dtype=jnp.float32), C, axis=1))
    bias_b = jnp.broadcast_to(bias.reshape(1, HC), (8, HC))
    out = pl.pallas_call(
        _fin_body,
        grid=(nblk,),
        in_specs=[
            pl.BlockSpec((brows, HC), lambda i: (i, 0)),
            pl.BlockSpec((brows, HC), lambda i: (i, 0)),
            pl.BlockSpec((brows, 16), lambda i: (i, 0)),
            pl.BlockSpec((brows, 16), lambda i: (i, 0)),
            pl.BlockSpec((8, HC), lambda i: (0, 0)),
            pl.BlockSpec((16, HC), lambda i: (0, 0)),
        ],
        out_specs=pl.BlockSpec((brows, HC), lambda i: (i, 0)),
        out_shape=jax.ShapeDtypeStruct((npad, HC), jnp.float32),
    )(num0, num1, den0, den1, bias_b, sel)
    return out[:n]


# speedup vs baseline: 61.0902x; 1.0071x over previous
import functools

import jax
import jax.numpy as jnp
from jax import lax
from jax.experimental import pallas as pl
from jax.experimental.pallas import tpu as pltpu
from jax.experimental.pallas import tpu_sc as plsc

NEG_SLOPE = 0.2
CB = 64
NCORES = 2
NSUB = 16
NTILES = NCORES * NSUB


def _proj_body(x_ref, wl_ref, wr_ref, xl_ref, xr_ref):
    xv = x_ref[...]
    xl_ref[...] = jnp.dot(xv, wl_ref[...], preferred_element_type=jnp.float32)
    xr_ref[...] = jnp.dot(xv, wr_ref[...], preferred_element_type=jnp.float32)


def _fin_body(n0_ref, n1_ref, d0_ref, d1_ref, b_ref, s_ref, o_ref):
    num = n0_ref[...] + n1_ref[...]
    den = d0_ref[...] + d1_ref[...]
    recip = 1.0 / (den + 1e-16)
    denx = jnp.dot(recip, s_ref[...], preferred_element_type=jnp.float32)
    o_ref[...] = jnp.maximum(num * denx + b_ref[...][0:1, :], 0.0)


def _make_sc_kernel(npad, epad, H, C):
    HC = H * C
    ept = epad // NTILES
    HB = CB // 2
    nh = ept // HB
    rps = npad // NSUB

    mesh = plsc.VectorSubcoreMesh(
        core_axis_name="c", subcore_axis_name="s",
        num_cores=NCORES, num_subcores=NSUB)

    @functools.partial(
        pl.kernel,
        out_type=[
            jax.ShapeDtypeStruct((npad, HC), jnp.float32),
            jax.ShapeDtypeStruct((npad, HC), jnp.float32),
            jax.ShapeDtypeStruct((npad // 8, 128), jnp.float32),
            jax.ShapeDtypeStruct((npad // 8, 128), jnp.float32),
        ],
        mesh=mesh,
        compiler_params=pltpu.CompilerParams(needs_layout_passes=False),
        scratch_types=[
            pltpu.VMEM((HC,), jnp.float32),
            pltpu.VMEM((HB,), jnp.int32),
            pltpu.VMEM((HB,), jnp.int32),
            pltpu.VMEM((HB,), jnp.int32),
            pltpu.VMEM((HB,), jnp.int32),
            pltpu.VMEM((HB,), jnp.int32),
            pltpu.VMEM((HB,), jnp.int32),
            pltpu.VMEM((HB, HC), jnp.float32),
            pltpu.VMEM((HB, HC), jnp.float32),
            pltpu.VMEM((HB, HC), jnp.float32),
            pltpu.VMEM((HB, HC), jnp.float32),
            pltpu.VMEM_SHARED((npad, HC), jnp.float32),
            pltpu.VMEM_SHARED((npad // 8, 128), jnp.float32),
            pltpu.SemaphoreType.DMA,
            pltpu.SemaphoreType.DMA,
            pltpu.SemaphoreType.DMA,
        ],
    )
    def sc_kernel(xl_h, xr_h, src_h, dst_h, att_h,
                  num0_o, num1_o, den0_o, den1_o,
                  att_v, src_a, dst_a, pk_a, src_b, dst_b, pk_b,
                  xla, xra, xlb, xrb, num_sh, den_sh, sema, semb, semc):
        cid = lax.axis_index("c")
        sid = lax.axis_index("s")
        r0 = sid * rps
        pr0 = sid * (rps // 8)
        iota = lax.iota(jnp.int32, 16)
        zero16 = jnp.zeros((16,), jnp.float32)
        nv = HC // 16

        pltpu.sync_copy(att_h, att_v)
        for r in range(8):
            for c8 in range(nv):
                xla[r, pl.ds(c8 * 16, 16)] = zero16

        def zinit(t, carry):
            pltpu.sync_copy(xla.at[pl.ds(0, 8)], num_sh.at[pl.ds(r0 + t * 8, 8)])
            return carry
        lax.fori_loop(0, rps // 8, zinit, 0)

        def zinit2(t, carry):
            pltpu.sync_copy(xla.at[pl.ds(0, 8)], den_sh.at[pl.ds(pr0 + t * 8, 8)])
            return carry
        lax.fori_loop(0, rps // 64, zinit2, 0)

        plsc.subcore_barrier()
        tid = cid * NSUB + sid
        ebase0 = tid * ept
        attv = [att_v[pl.ds(k * 16, 16)] for k in range(nv)]
        kph = C // 16

        def issue(idx, sv, dv, xlr, xrr, sem):
            eb = ebase0 + idx * HB
            pltpu.sync_copy(src_h.at[pl.ds(eb, HB)], sv)
            pltpu.sync_copy(dst_h.at[pl.ds(eb, HB)], dv)
            pltpu.async_copy(xl_h.at[sv], xlr, sem)
            pltpu.async_copy(xr_h.at[dv], xrr, sem)

        def drain(xlr, xrr, sem):
            pltpu.make_async_copy(xl_h.at[pl.ds(0, HB)], xlr, sem).wait()
            pltpu.make_async_copy(xl_h.at[pl.ds(0, HB)], xrr, sem).wait()

        def process(sv, dv, pkv, xlr, xrr):
            for g16 in range(HB // 16):
                dvv = dv[pl.ds(g16 * 16, 16)]
                pkv[pl.ds(g16 * 16, 16)] = lax.shift_right_logical(dvv, 3)

            def edge(r, carry):
                rv = jnp.full((16,), r, jnp.int32)
                roww = zero16
                for h in range(H):
                    acc = zero16
                    xsl = []
                    for k in range(kph):
                        sl = pl.ds((h * kph + k) * 16, 16)
                        xv = xlr[r, sl]
                        xsl.append(xv)
                        v = xv + xrr[r, sl]
                        v = jnp.maximum(v, NEG_SLOPE * v)
                        acc = acc + attv[h * kph + k] * v
                    w = jnp.exp(jnp.full((16,), jnp.sum(acc)))
                    for k in range(kph):
                        sl = pl.ds((h * kph + k) * 16, 16)
                        xlr[r, sl] = xsl[k] * w
                    roww = jnp.where(iota == h, w, roww)
                dsplat = plsc.load_gather(dv, [rv])
                for s8 in range(nv):
                    xrr[r, pl.ds(s8 * 16, 16)] = zero16
                colv = (dsplat & 7) * 16 + iota
                plsc.store_scatter(xrr, [rv, colv], roww)
                return carry
            plsc.parallel_loop(0, HB, 1, unroll=16, carry=jnp.int32(0))(edge)

            s1 = pltpu.async_copy(xlr, num_sh.at[dv], semc, add=True)
            s2 = pltpu.async_copy(xrr, den_sh.at[pkv], semc, add=True)
            s1.wait()
            s2.wait()

        issue(0, src_a, dst_a, xla, xra, sema)

        def chunk(gc, carry):
            i0 = gc * 2
            issue(i0 + 1, src_b, dst_b, xlb, xrb, semb)
            drain(xla, xra, sema)
            process(src_a, dst_a, pk_a, xla, xra)

            @pl.when(i0 + 2 < nh)
            def _():
                issue(i0 + 2, src_a, dst_a, xla, xra, sema)
            drain(xlb, xrb, semb)
            process(src_b, dst_b, pk_b, xlb, xrb)
            return carry

        lax.fori_loop(0, nh // 2, chunk, 0)
        plsc.subcore_barrier()

        def copy_out(num_o, den_o):
            def cp(t, carry):
                rr = r0 + t * HB
                pltpu.sync_copy(num_sh.at[pl.ds(rr, HB)], xla)
                pltpu.sync_copy(xla, num_o.at[pl.ds(rr, HB)])
                return carry
            lax.fori_loop(0, rps // HB, cp, 0)

            def cpd(t, carry):
                prr = pr0 + t * 8
                pltpu.sync_copy(den_sh.at[pl.ds(prr, 8)], xra.at[pl.ds(0, 8)])
                pltpu.sync_copy(xra.at[pl.ds(0, 8)], den_o.at[pl.ds(prr, 8)])
                return carry
            lax.fori_loop(0, rps // 64, cpd, 0)

        @pl.when(cid == 0)
        def _():
            copy_out(num0_o, den0_o)

        @pl.when(cid == 1)
        def _():
            copy_out(num1_o, den1_o)

    return sc_kernel


def kernel(x, edge_index, W_l, W_r, att, bias):
    n, in_dim = x.shape
    H, C = att.shape
    HC = H * C
    e = edge_index.shape[1]
    etot = e + n
    npad = ((n + 1 + 1023) // 1024) * 1024
    epad = ((etot + CB * NTILES - 1) // (CB * NTILES)) * (CB * NTILES)

    x_pad = jnp.zeros((npad, in_dim), jnp.float32).at[:n].set(x)
    loop = jnp.arange(n, dtype=jnp.int32)
    padi = jnp.full((epad - etot,), n, dtype=jnp.int32)
    src_all = jnp.concatenate([edge_index[0], loop, padi])
    dst_all = jnp.concatenate([edge_index[1], loop, padi])
    att_flat = att.reshape(HC)

    nblk = 8
    brows = npad // nblk
    xl, xr = pl.pallas_call(
        _proj_body,
        grid=(nblk,),
        in_specs=[
            pl.BlockSpec((brows, in_dim), lambda i: (i, 0)),
            pl.BlockSpec((in_dim, HC), lambda i: (0, 0)),
            pl.BlockSpec((in_dim, HC), lambda i: (0, 0)),
        ],
        out_specs=[
            pl.BlockSpec((brows, HC), lambda i: (i, 0)),
            pl.BlockSpec((brows, HC), lambda i: (i, 0)),
        ],
        out_shape=[
            jax.ShapeDtypeStruct((npad, HC), jnp.float32),
            jax.ShapeDtypeStruct((npad, HC), jnp.float32),
        ],
    )(x_pad, W_l, W_r)

    sc_kernel = _make_sc_kernel(npad, epad, H, C)
    num0, num1, den0, den1 = sc_kernel(
        xl, xr, src_all, dst_all, att_flat)
    den0 = den0.reshape(npad, 16)
    den1 = den1.reshape(npad, 16)

    sel = jnp.zeros((16, HC), jnp.float32).at[:H].set(
        jnp.repeat(jnp.eye(H, dtype=jnp.float32), C, axis=1))
    bias_b = jnp.broadcast_to(bias.reshape(1, HC), (8, HC))
    out = pl.pallas_call(
        _fin_body,
        grid=(nblk,),
        in_specs=[
            pl.BlockSpec((brows, HC), lambda i: (i, 0)),
            pl.BlockSpec((brows, HC), lambda i: (i, 0)),
            pl.BlockSpec((brows, 16), lambda i: (i, 0)),
            pl.BlockSpec((brows, 16), lambda i: (i, 0)),
            pl.BlockSpec((8, HC), lambda i: (0, 0)),
            pl.BlockSpec((16, HC), lambda i: (0, 0)),
        ],
        out_specs=pl.BlockSpec((brows, HC), lambda i: (i, 0)),
        out_shape=jax.ShapeDtypeStruct((npad, HC), jnp.float32),
    )(num0, num1, den0, den1, bias_b, sel)
    return out[:n]
